# baseline (speedup 1.0000x reference)
import functools

import jax
import jax.numpy as jnp
from jax.experimental import pallas as pl

_N0, _N1, _N2, _C, _NC = 10000, 320000, 100000, 128, 10


def _colsum_body(s_ref, w_ref, l_ref, o_ref):
    @pl.when(pl.program_id(0) == 0)
    def _():
        o_ref[...] = jnp.zeros_like(o_ref)

    h = jnp.maximum(
        jnp.dot(s_ref[...], w_ref[...], preferred_element_type=jnp.float32), 0.0
    )
    cs = jnp.sum(h, axis=0, keepdims=True)
    o_ref[...] += jnp.dot(cs, l_ref[...], preferred_element_type=jnp.float32)


def _head(s, w, lin_w, block):
    n = s.shape[0]
    grid = n // block
    return pl.pallas_call(
        _colsum_body,
        grid=(grid,),
        in_specs=[
            pl.BlockSpec((block, _C), lambda i: (i, 0)),
            pl.BlockSpec((_C, _C), lambda i: (0, 0)),
            pl.BlockSpec((_C, _NC), lambda i: (0, 0)),
        ],
        out_specs=pl.BlockSpec((1, _NC), lambda i: (0, 0)),
        out_shape=jax.ShapeDtypeStruct((1, _NC), jnp.float32),
    )(s, w, lin_w)


def kernel(x_0, x_a_1, x_b_1, n00_indices, n00_values, n12_indices, n12_values,
           conv1_w, conv2_w, conv3_w, lin1_w, lin1_b, lin2_w, lin2_b, lin3_w, lin3_b):
    i00 = n00_indices.astype(jnp.int32)
    i12 = n12_indices.astype(jnp.int32)

    r0 = jax.nn.relu(x_0)
    ra = jax.nn.relu(x_a_1)
    rb = jax.nn.relu(x_b_1)

    s0 = jax.ops.segment_sum(
        n00_values[:, None] * jnp.take(r0, i00[1], axis=0), i00[0], num_segments=_N0)
    sa = jax.ops.segment_sum(
        n12_values[:, None] * jnp.take(ra, i12[1], axis=0), i12[0], num_segments=_N2)
    sb = jax.ops.segment_sum(
        n12_values[:, None] * jnp.take(rb, i12[1], axis=0), i12[0], num_segments=_N2)

    oa = _head(sa, conv2_w, lin1_w, 2000)[0] / _N2
    ob = _head(sb, conv3_w, lin2_w, 2000)[0] / _N2
    o0 = _head(s0, conv1_w, lin3_w, 2000)[0] / _N0

    return oa + ob + o0 + lin1_b + lin2_b + lin3_b


# speedup vs baseline: 2.7106x; 2.7106x over previous
import jax
import jax.numpy as jnp
from jax import lax
from jax.experimental import pallas as pl
from jax.experimental.pallas import tpu as pltpu
from jax.experimental.pallas import tpu_sc as plsc

_N0, _N1, _N2, _C, _NCLS = 10000, 320000, 100000, 128, 10
_E00, _E12 = 320000, 400000

_NCORE, _NSUB, _NW = 2, 16, 32
_CHUNK = 5888
_N2P = 105984
_N0P = 11776
_GB = 64
_SEG = 4096
_E12P = 400384
_EPT12 = _E12P // _NW
_EPT00 = _E00 // _NW


def _sc_body(xa, xb, x0, r12, c12, v12, r00, c00, v00,
             outa, outb, out0,
             erow, ecol, evalv, cidx, crow, cval, g0, g1, zbuf, acc,
             sem0, sem1):
    cid = lax.axis_index("c")
    sid = lax.axis_index("s")
    wid = sid * _NCORE + cid

    iota = lax.iota(jnp.int32, 16)
    zi = jnp.zeros((16,), jnp.int32)
    zf = jnp.zeros((16,), jnp.float32)
    padcol = wid * _GB + iota

    def _zb(i, carry):
        for j in range(8):
            zbuf[i, pl.ds(j * 16, 16)] = zf
        return carry
    lax.fori_loop(0, 32, _zb, 0)

    def process(x_ref, r_ref, c_ref, v_ref, out_ref, ept, nseg):
        base = wid * ept
        pltpu.sync_copy(r_ref.at[pl.ds(base, ept)], erow.at[pl.ds(0, ept)])
        pltpu.sync_copy(c_ref.at[pl.ds(base, ept)], ecol.at[pl.ds(0, ept)])
        pltpu.sync_copy(v_ref.at[pl.ds(base, ept)], evalv.at[pl.ds(0, ept)])

        nfull, tail = ept // _SEG, ept % _SEG
        stripe = _CHUNK // _NSUB
        sbase = sid * stripe
        npass = nseg // _CHUNK

        def one_seg(sb, seg, lo, hi):
            def scan_body(i, cnt):
                r = erow[pl.ds(sb + i * 16, 16)]
                c = ecol[pl.ds(sb + i * 16, 16)]
                v = evalv[pl.ds(sb + i * 16, 16)]
                m = (r >= lo) & (r < hi)
                mi = jnp.where(m, 1, 0).astype(jnp.int32)
                pos = plsc.cumsum(mi) + (cnt - 1)
                pr = lax.shift_right_logical(pos, 6)
                pc = lax.bitwise_and(pos, _GB - 1)
                plsc.store_scatter(cidx, [pr, pc], c, mask=m)
                plsc.store_scatter(cval, [pr, pc], v, mask=m)
                plsc.store_scatter(crow, [pr, pc], r - lo, mask=m)
                return cnt + jnp.sum(mi)
            cnt = lax.fori_loop(0, seg // 16, scan_body, jnp.int32(0))

            nblk = (cnt + (_GB - 1)) // _GB
            lim = nblk * _GB
            for j in range(4):
                pp = cnt + (j * 16) + iota
                pm = pp < lim
                ppr = lax.shift_right_logical(pp, 6)
                ppc = lax.bitwise_and(pp, _GB - 1)
                plsc.store_scatter(cidx, [ppr, ppc], padcol, mask=pm)
                plsc.store_scatter(cval, [ppr, ppc], zf, mask=pm)
                plsc.store_scatter(crow, [ppr, ppc], zi, mask=pm)

            @pl.when(nblk > 0)
            def _():
                pltpu.async_copy(x_ref.at[cidx.at[0]], g0, sem0)

            def do_block(k, g, sem, other_g, other_sem):
                @pl.when(k < nblk)
                def _():
                    pltpu.make_async_copy(
                        x_ref.at[cidx.at[k]], g, sem).wait()

                    @pl.when(k + 1 < nblk)
                    def _():
                        pltpu.async_copy(x_ref.at[cidx.at[k + 1]],
                                         other_g, other_sem)

                    def ebody(e, carry):
                        vv = plsc.load_gather(cval, [zi + k, zi + e])
                        for j in range(8):
                            gv = g[e, pl.ds(j * 16, 16)]
                            g[e, pl.ds(j * 16, 16)] = (
                                jnp.maximum(gv, 0.0) * vv)
                        return carry
                    lax.fori_loop(0, _GB, ebody, 0)
                    pltpu.sync_copy(g, acc.at[crow.at[k]], add=True)

            def blk_pair(kk, carry):
                do_block(kk * 2, g0, sem0, g1, sem1)
                do_block(kk * 2 + 1, g1, sem1, g0, sem0)
                return carry
            lax.fori_loop(0, (nblk + 1) // 2, blk_pair, 0)

        def one_pass(p, carry):
            lo = p * _CHUNK
            hi = lo + _CHUNK

            nz_full, rem = stripe // 32, stripe % 32
            for j in range(nz_full):
                pltpu.sync_copy(zbuf, acc.at[pl.ds(sbase + j * 32, 32)])
            if rem:
                pltpu.sync_copy(zbuf.at[pl.ds(0, rem)],
                                acc.at[pl.ds(sbase + nz_full * 32, rem)])
            plsc.subcore_barrier()

            def seg_loop(s, carry2):
                one_seg(s * _SEG, _SEG, lo, hi)
                return carry2
            lax.fori_loop(0, nfull, seg_loop, 0)
            if tail:
                one_seg(nfull * _SEG, tail, lo, hi)

            plsc.subcore_barrier()
            pltpu.sync_copy(acc.at[pl.ds(sbase, stripe)],
                            out_ref.at[cid, pl.ds(lo + sbase, stripe)])
            return carry
        lax.fori_loop(0, npass, one_pass, 0)

    process(xa, r12, c12, v12, outa, _EPT12, _N2P)
    process(xb, r12, c12, v12, outb, _EPT12, _N2P)
    process(x0, r00, c00, v00, out0, _EPT00, _N0P)


_sc_segsum = pl.kernel(
    _sc_body,
    out_type=(
        jax.ShapeDtypeStruct((_NCORE, _N2P, _C), jnp.float32),
        jax.ShapeDtypeStruct((_NCORE, _N2P, _C), jnp.float32),
        jax.ShapeDtypeStruct((_NCORE, _N0P, _C), jnp.float32),
    ),
    mesh=plsc.VectorSubcoreMesh(
        core_axis_name="c", subcore_axis_name="s",
        num_cores=_NCORE, num_subcores=_NSUB),
    scratch_types=[
        pltpu.VMEM((_EPT12,), jnp.int32),
        pltpu.VMEM((_EPT12,), jnp.int32),
        pltpu.VMEM((_EPT12,), jnp.float32),
        pltpu.VMEM((_SEG // _GB, _GB), jnp.int32),
        pltpu.VMEM((_SEG // _GB, _GB), jnp.int32),
        pltpu.VMEM((_SEG // _GB, _GB), jnp.float32),
        pltpu.VMEM((_GB, _C), jnp.float32),
        pltpu.VMEM((_GB, _C), jnp.float32),
        pltpu.VMEM((32, _C), jnp.float32),
        pltpu.VMEM_SHARED((_CHUNK, _C), jnp.float32),
        pltpu.SemaphoreType.DMA,
        pltpu.SemaphoreType.DMA,
    ],
    compiler_params=pltpu.CompilerParams(needs_layout_passes=False),
    name="cxn_sc_segsum",
)


def _head_body(p0_ref, p1_ref, w_ref, l_ref, o_ref):
    @pl.when(pl.program_id(0) == 0)
    def _():
        o_ref[...] = jnp.zeros_like(o_ref)

    s = p0_ref[0] + p1_ref[0]
    h = jnp.maximum(
        jnp.dot(s, w_ref[...], preferred_element_type=jnp.float32), 0.0)
    cs = jnp.sum(h, axis=0, keepdims=True)
    o_ref[...] += jnp.dot(cs, l_ref[...], preferred_element_type=jnp.float32)


def _head(sp, w, lin_w, block):
    n = sp.shape[1]
    return pl.pallas_call(
        _head_body,
        grid=(n // block,),
        in_specs=[
            pl.BlockSpec((1, block, _C), lambda i: (0, i, 0)),
            pl.BlockSpec((1, block, _C), lambda i: (1, i, 0)),
            pl.BlockSpec((_C, _C), lambda i: (0, 0)),
            pl.BlockSpec((_C, _NCLS), lambda i: (0, 0)),
        ],
        out_specs=pl.BlockSpec((1, _NCLS), lambda i: (0, 0)),
        out_shape=jax.ShapeDtypeStruct((1, _NCLS), jnp.float32),
    )(sp, sp, w, lin_w)


def kernel(x_0, x_a_1, x_b_1, n00_indices, n00_values, n12_indices, n12_values,
           conv1_w, conv2_w, conv3_w, lin1_w, lin1_b, lin2_w, lin2_b, lin3_w, lin3_b):
    i00 = n00_indices.astype(jnp.int32)
    i12 = n12_indices.astype(jnp.int32)
    pad = _E12P - _E12
    r12 = jnp.concatenate([i12[0], jnp.full((pad,), _N2P, jnp.int32)])
    c12 = jnp.concatenate([i12[1], jnp.zeros((pad,), jnp.int32)])
    v12 = jnp.concatenate([n12_values, jnp.zeros((pad,), jnp.float32)])

    sa_p, sb_p, s0_p = _sc_segsum(
        x_a_1, x_b_1, x_0, r12, c12, v12, i00[0], i00[1], n00_values)

    oa = _head(sa_p, conv2_w, lin1_w, 4416)[0] / _N2
    ob = _head(sb_p, conv3_w, lin2_w, 4416)[0] / _N2
    o0 = _head(s0_p, conv1_w, lin3_w, 1472)[0] / _N0

    return oa + ob + o0 + lin1_b + lin2_b + lin3_b
